# Initial kernel scaffold; baseline (speedup 1.0000x reference)
import jax
import jax.numpy as jnp
from jax.experimental import pallas as pl


def kernel(roi_features, union_features, rel_pair_idxs, obj_labels, W_in, W_msg, W_upd, W_pair, W_fuse, W_objproj, W_objcls, b_objcls, W_relcls, b_relcls, freq_bias):
    raise NotImplementedError("write your pallas kernel here")


# speedup vs baseline: 3.5937x; 3.5937x over previous
import dataclasses
import functools

import jax
import jax.numpy as jnp
from jax import lax
from jax.experimental import pallas as pl
from jax.experimental.pallas import tpu as pltpu
from jax.experimental.pallas import tpu_sc as plsc

N_OBJ = 10000
N_REL = 160000
IN_DIM = 256
HID = 512
POOL = 512
NUM_OBJ_CLS = 151
NUM_REL_CLS = 51

NC, NS, L = 2, 16, 16
N_PAD = 10112
ROWS_PER_TILE = N_PAD // NS
F32 = jnp.float32

PREC = jax.lax.Precision.DEFAULT

@functools.cache
def _mesh():
    return plsc.VectorSubcoreMesh(core_axis_name="c", subcore_axis_name="s")


def _sc_params():
    cp = pltpu.CompilerParams()
    if "needs_layout_passes" in pltpu.CompilerParams.__dataclass_fields__:
        cp = dataclasses.replace(cp, needs_layout_passes=False)
    return cp


def _sc_deg(obj32, zeros128):
    @functools.partial(
        pl.kernel, mesh=_mesh(),
        out_type=jax.ShapeDtypeStruct((NC, N_PAD, 128), F32),
        scratch_types=[
            pltpu.VMEM((125, 40), jnp.int32),
            pltpu.VMEM((40, 128), F32),
            pltpu.VMEM_SHARED((N_PAD, 128), F32),
        ],
    )
    def k(obj_hbm, z_hbm, out_hbm, idx_v, ones_v, deg_sh):
        c = lax.axis_index("c")
        s = lax.axis_index("s")
        w = c * NS + s
        pltpu.sync_copy(obj_hbm.at[w], idx_v)

        @pl.loop(0, 40)
        def _(i):
            for v in range(8):
                ones_v[i, pl.ds(v * 16, 16)] = jnp.full((16,), 1.0, F32)

        stripe = pl.ds(s * ROWS_PER_TILE, ROWS_PER_TILE)
        pltpu.sync_copy(z_hbm.at[stripe], deg_sh.at[stripe])
        plsc.subcore_barrier()

        @pl.loop(0, 125)
        def _(j):
            pltpu.sync_copy(ones_v, deg_sh.at[idx_v.at[j]], add=True)

        plsc.subcore_barrier()
        pltpu.sync_copy(deg_sh.at[stripe], out_hbm.at[c].at[stripe])

    return k(obj32, zeros128)


def _sc_agg(ab, subg, objg, zeros128):
    @functools.partial(
        pl.kernel, mesh=_mesh(),
        out_type=jax.ShapeDtypeStruct((4, N_PAD, 128), F32),
        scratch_types=[
            pltpu.VMEM((50, 40), jnp.int32),
            pltpu.VMEM((50, 40), jnp.int32),
            pltpu.VMEM((40, 128), F32),
            pltpu.VMEM((40, 128), F32),
            pltpu.VMEM((40, 128), F32),
            pltpu.VMEM((40, 128), F32),
            pltpu.VMEM((40, 128), F32),
            pltpu.VMEM((40, 128), F32),
            pltpu.SemaphoreType.DMA,
            pltpu.SemaphoreType.DMA,
            pltpu.SemaphoreType.DMA,
            pltpu.SemaphoreType.DMA,
            pltpu.VMEM_SHARED((N_PAD, 128), F32),
        ],
    )
    def k(ab_hbm, sub_hbm, obj_hbm, z_hbm, agg_hbm,
          sub_g, obj_g, ra0, ra1, rb0, rb1, mb0, mb1,
          sg0, sg1, ss0, ss1, agg_sh):
        c = lax.axis_index("c")
        s = lax.axis_index("s")
        stripe = pl.ds(s * ROWS_PER_TILE, ROWS_PER_TILE)
        ra = (ra0, ra1)
        rb = (rb0, rb1)
        mb = (mb0, mb1)
        sg = (sg0, sg1)
        ss = (ss0, ss1)

        for rnd in range(2):
            chunk = 2 * rnd + c
            a_src = ab_hbm.at[chunk]
            b_src = ab_hbm.at[4 + chunk]

            def gather_start(p, lb):
                pltpu.make_async_copy(a_src.at[sub_g.at[lb]], ra[p],
                                      sg[p]).start()
                pltpu.make_async_copy(b_src.at[obj_g.at[lb]], rb[p],
                                      sg[p]).start()

            def gather_wait(p):
                pltpu.make_async_copy(a_src.at[sub_g.at[0]], ra[p],
                                      sg[p]).wait()
                pltpu.make_async_copy(b_src.at[obj_g.at[0]], rb[p],
                                      sg[p]).wait()

            def compute(p):
                @pl.loop(0, 40)
                def _(i):
                    for v in range(8):
                        sl = pl.ds(v * 16, 16)
                        mb[p][i, sl] = jnp.maximum(ra[p][i, sl] + rb[p][i, sl],
                                                   0.0)

            def scatter_start(p, lb):
                pltpu.make_async_copy(mb[p], agg_sh.at[obj_g.at[lb]],
                                      ss[p]).start(add=True)

            def scatter_wait(p):
                pltpu.make_async_copy(mb[p], agg_sh.at[obj_g.at[0]],
                                      ss[p]).wait()

            pltpu.sync_copy(z_hbm.at[stripe], agg_sh.at[stripe])
            plsc.subcore_barrier()

            for g in range(5):
                pltpu.sync_copy(sub_hbm.at[s].at[g], sub_g)
                pltpu.sync_copy(obj_hbm.at[s].at[g], obj_g)
                gather_start(0, 0)
                gather_start(1, 1)

                @pl.loop(0, 24)
                def _(jj):
                    for p in range(2):
                        lb = 2 * jj + p
                        gather_wait(p)

                        @pl.when(jj > 0)
                        def _():
                            scatter_wait(p)

                        compute(p)
                        scatter_start(p, lb)
                        gather_start(p, lb + 2)

                for p in range(2):
                    lb = 48 + p
                    gather_wait(p)
                    scatter_wait(p)
                    compute(p)
                    scatter_start(p, lb)
                scatter_wait(0)
                scatter_wait(1)

            plsc.subcore_barrier()
            pltpu.sync_copy(agg_sh.at[stripe],
                            agg_hbm.at[chunk].at[stripe])
            plsc.subcore_barrier()

    return k(ab, subg, objg, zeros128)


def _sc_pair(p2, subpp, objpp):
    n_pp = 32 * 160 * 32

    @functools.partial(
        pl.kernel, mesh=_mesh(),
        out_type=jax.ShapeDtypeStruct((n_pp, POOL), F32),
        scratch_types=[
            pltpu.VMEM((160, 32), jnp.int32),
            pltpu.VMEM((160, 32), jnp.int32),
            pltpu.VMEM((32, POOL), F32),
            pltpu.VMEM((32, POOL), F32),
            pltpu.VMEM((32, POOL), F32),
            pltpu.VMEM((32, POOL), F32),
            pltpu.VMEM((32, POOL), F32),
            pltpu.SemaphoreType.DMA,
            pltpu.SemaphoreType.DMA,
            pltpu.SemaphoreType.DMA,
        ],
    )
    def k(p_hbm, sub_hbm, obj_hbm, out_hbm,
          sub_v, obj_v, ra0, ra1, rb0, rb1, mb, sg0, sg1, sw):
        c = lax.axis_index("c")
        s = lax.axis_index("s")
        w = c * NS + s
        pltpu.sync_copy(sub_hbm.at[w], sub_v)
        pltpu.sync_copy(obj_hbm.at[w], obj_v)
        base = w * 5120
        ra = (ra0, ra1)
        rb = (rb0, rb1)
        sg = (sg0, sg1)

        def gather_start(p, b):
            pltpu.make_async_copy(p_hbm.at[0].at[sub_v.at[b]], ra[p],
                                  sg[p]).start()
            pltpu.make_async_copy(p_hbm.at[1].at[obj_v.at[b]], rb[p],
                                  sg[p]).start()

        def gather_wait(p):
            pltpu.make_async_copy(p_hbm.at[0].at[sub_v.at[0]], ra[p],
                                  sg[p]).wait()
            pltpu.make_async_copy(p_hbm.at[1].at[obj_v.at[0]], rb[p],
                                  sg[p]).wait()

        def compute(p):
            @pl.loop(0, 32)
            def _(i):
                for v in range(POOL // 16):
                    sl = pl.ds(v * 16, 16)
                    mb[i, sl] = ra[p][i, sl] + rb[p][i, sl]

        def write_start(b):
            pltpu.make_async_copy(
                mb, out_hbm.at[pl.ds(base + b * 32, 32)], sw).start()

        def write_wait():
            pltpu.make_async_copy(
                mb, out_hbm.at[pl.ds(base, 32)], sw).wait()

        gather_start(0, 0)
        gather_start(1, 1)

        @pl.loop(0, 80)
        def _(j):
            gather_wait(0)

            @pl.when(j > 0)
            def _():
                write_wait()

            compute(0)
            write_start(2 * j)

            @pl.when(j < 79)
            def _():
                gather_start(0, 2 * j + 2)

            gather_wait(1)
            write_wait()
            compute(1)
            write_start(2 * j + 1)

            @pl.when(j < 79)
            def _():
                gather_start(1, 2 * j + 3)

        write_wait()

    return k(p2, subpp, objpp)


def _sc_freq(subp, objp, obj_labels, freq_pad):
    n_pad = 32 * 64 * 80

    @functools.partial(
        pl.kernel, mesh=_mesh(), compiler_params=_sc_params(),
        out_type=jax.ShapeDtypeStruct((n_pad, 128), F32),
        scratch_types=[
            pltpu.VMEM((N_OBJ,), jnp.int32),
            pltpu.VMEM((64, 80), jnp.int32),
            pltpu.VMEM((64, 80), jnp.int32),
            pltpu.VMEM((64, 80), jnp.int32),
            pltpu.VMEM((80, 128), F32),
            pltpu.VMEM((80, 128), F32),
            pltpu.VMEM((80, 128), F32),
            pltpu.VMEM((80, 128), F32),
            pltpu.SemaphoreType.DMA,
            pltpu.SemaphoreType.DMA,
            pltpu.SemaphoreType.DMA,
            pltpu.SemaphoreType.DMA,
            pltpu.SemaphoreType.DMA,
            pltpu.SemaphoreType.DMA,
            pltpu.SemaphoreType.DMA,
            pltpu.SemaphoreType.DMA,
        ],
    )
    def k(sub_hbm, obj_hbm, lbl_hbm, fq_hbm, out_hbm,
          lbl_v, sub_v, obj_v, pair_v, rv0, rv1, rv2, rv3,
          sg0, sg1, sg2, sg3, sw0, sw1, sw2, sw3):
        c = lax.axis_index("c")
        s = lax.axis_index("s")
        w = c * NS + s
        pltpu.sync_copy(lbl_hbm, lbl_v)
        pltpu.sync_copy(sub_hbm.at[w], sub_v)
        pltpu.sync_copy(obj_hbm.at[w], obj_v)
        rv = (rv0, rv1, rv2, rv3)
        sg = (sg0, sg1, sg2, sg3)
        sw = (sw0, sw1, sw2, sw3)

        @pl.loop(0, 64)
        def _(j):
            for v in range(5):
                sl = pl.ds(v * 16, 16)
                ls = plsc.load_gather(lbl_v, [sub_v[j, sl]])
                lo = plsc.load_gather(lbl_v, [obj_v[j, sl]])
                pair_v[j, sl] = ls * NUM_OBJ_CLS + lo

        def gather_start(p, b):
            pltpu.make_async_copy(fq_hbm.at[pair_v.at[b]], rv[p],
                                  sg[p]).start()

        def gather_wait(p):
            pltpu.make_async_copy(fq_hbm.at[pair_v.at[0]], rv[p],
                                  sg[p]).wait()

        def write_start(p, b):
            pltpu.make_async_copy(
                rv[p], out_hbm.at[pl.ds(w * 5120 + b * 80, 80)], sw[p]).start()

        def write_wait(p):
            pltpu.make_async_copy(
                rv[p], out_hbm.at[pl.ds(w * 5120, 80)], sw[p]).wait()

        for p in range(4):
            gather_start(p, p)

        @pl.loop(0, 16)
        def _(j):
            for p in range(4):
                b = 4 * j + p
                gather_wait(p)
                write_start(p, b)

                @pl.when(j < 15)
                def _():
                    write_wait(p)
                    gather_start(p, b + 4)

        for p in range(4):
            write_wait(p)

    return k(subp, objp, obj_labels, freq_pad)


def _dot(a, b):
    return jnp.dot(a, b, preferred_element_type=F32, precision=PREC)


def _tc_in_msg(roi, w_in, w_cat):
    def body(x_ref, wi_ref, wc_ref, oh_ref, oab_ref):
        h = jax.nn.relu(_dot(x_ref[...], wi_ref[...]))
        oh_ref[...] = h
        ab = _dot(h, wc_ref[...])
        for cidx in range(8):
            oab_ref[cidx] = ab[:, cidx * 128:(cidx + 1) * 128]

    return pl.pallas_call(
        body,
        grid=(10,),
        in_specs=[pl.BlockSpec((1000, IN_DIM), lambda i: (i, 0)),
                  pl.BlockSpec((IN_DIM, HID), lambda i: (0, 0)),
                  pl.BlockSpec((HID, 2 * HID), lambda i: (0, 0))],
        out_specs=[pl.BlockSpec((1000, HID), lambda i: (i, 0)),
                   pl.BlockSpec((8, 1000, 128), lambda i: (0, i, 0))],
        out_shape=[jax.ShapeDtypeStruct((N_OBJ, HID), F32),
                   jax.ShapeDtypeStruct((8, N_OBJ, 128), F32)],
    )(roi, w_in, w_cat)


def _tc_upd_proj(h, agg, deg2, w_upd, w_cat, nchunks):
    cw = 2 * HID // nchunks

    def body(h_ref, a_ref, d_ref, w_ref, wc_ref, oh_ref, op_ref):
        deg = d_ref[0, :, 0:1] + d_ref[1, :, 0:1]
        inv = 1.0 / jnp.maximum(deg, 1.0)
        a_full = jnp.concatenate([a_ref[cidx] for cidx in range(4)], axis=1)
        upd = _dot(a_full * inv, w_ref[...])
        hn = jax.nn.relu(h_ref[...] + upd)
        oh_ref[...] = hn
        pr = _dot(hn, wc_ref[...])
        for cidx in range(nchunks):
            op_ref[cidx] = pr[:, cidx * cw:(cidx + 1) * cw]

    return pl.pallas_call(
        body,
        grid=(10,),
        in_specs=[pl.BlockSpec((1000, HID), lambda i: (i, 0)),
                  pl.BlockSpec((4, 1000, 128), lambda i: (0, i, 0)),
                  pl.BlockSpec((2, 1000, 128), lambda i: (0, i, 0)),
                  pl.BlockSpec((HID, HID), lambda i: (0, 0)),
                  pl.BlockSpec((HID, 2 * HID), lambda i: (0, 0))],
        out_specs=[pl.BlockSpec((1000, HID), lambda i: (i, 0)),
                   pl.BlockSpec((nchunks, 1000, cw), lambda i: (0, i, 0))],
        out_shape=[jax.ShapeDtypeStruct((N_OBJ, HID), F32),
                   jax.ShapeDtypeStruct((nchunks, N_OBJ, cw), F32)],
    )(h, agg, deg2, w_upd, w_cat)


def _tc_rel(union, relh, fb, w_fuse, w_rel, b_rel):
    def body(u_ref, r_ref, f_ref, wf_ref, wr_ref, b_ref, o_ref):
        t = jax.nn.relu(_dot(u_ref[...], wf_ref[...])
                        + jax.nn.relu(r_ref[...]))
        o_ref[...] = _dot(t, wr_ref[...]) + b_ref[...] + f_ref[:, :64]

    return pl.pallas_call(
        body,
        grid=(160,),
        in_specs=[pl.BlockSpec((1000, POOL), lambda i: (i, 0)),
                  pl.BlockSpec((1000, POOL), lambda i: (i, 0)),
                  pl.BlockSpec((1000, 128), lambda i: (i, 0)),
                  pl.BlockSpec((POOL, POOL), lambda i: (0, 0)),
                  pl.BlockSpec((POOL, 64), lambda i: (0, 0)),
                  pl.BlockSpec((1, 64), lambda i: (0, 0))],
        out_specs=pl.BlockSpec((1000, 64), lambda i: (i, 0)),
        out_shape=jax.ShapeDtypeStruct((N_REL, 64), F32),
    )(union, relh, fb, w_fuse, w_rel, b_rel)


def _tc_obj(h, w_proj, w_cls, b_cls):
    def body(h_ref, wp_ref, wc_ref, b_ref, o_ref):
        f = jax.nn.relu(_dot(h_ref[...], wp_ref[...]))
        o_ref[...] = _dot(f, wc_ref[...]) + b_ref[...]

    return pl.pallas_call(
        body,
        grid=(10,),
        in_specs=[pl.BlockSpec((1000, HID), lambda i: (i, 0)),
                  pl.BlockSpec((HID, POOL), lambda i: (0, 0)),
                  pl.BlockSpec((POOL, 256), lambda i: (0, 0)),
                  pl.BlockSpec((1, 256), lambda i: (0, 0))],
        out_specs=pl.BlockSpec((1000, 256), lambda i: (i, 0)),
        out_shape=jax.ShapeDtypeStruct((N_OBJ, 256), F32),
    )(h, w_proj, w_cls, b_cls)


def kernel(roi_features, union_features, rel_pair_idxs, obj_labels, W_in,
           W_msg, W_upd, W_pair, W_fuse, W_objproj, W_objcls, b_objcls,
           W_relcls, b_relcls, freq_bias):
    sub_idx = rel_pair_idxs[:, 0]
    obj_idx = rel_pair_idxs[:, 1]

    subg = sub_idx.reshape(NS, 5, 50, 40)
    objg = obj_idx.reshape(NS, 5, 50, 40)
    sub32 = sub_idx.reshape(NC * NS, 125, 40)
    obj32 = obj_idx.reshape(NC * NS, 125, 40)
    pad = 32 * 64 * 80 - N_REL
    subp = jnp.concatenate(
        [sub_idx, jnp.zeros((pad,), jnp.int32)]).reshape(32, 64, 80)
    objp = jnp.concatenate(
        [obj_idx, jnp.zeros((pad,), jnp.int32)]).reshape(32, 64, 80)
    subpp = jnp.concatenate(
        [sub_idx, jnp.zeros((pad,), jnp.int32)]).reshape(32, 160, 32)
    objpp = jnp.concatenate(
        [obj_idx, jnp.zeros((pad,), jnp.int32)]).reshape(32, 160, 32)

    zeros128 = jnp.zeros((N_PAD, 128), F32)

    w_msg_cat = jnp.concatenate([W_msg[:HID], W_msg[HID:]], axis=1)
    w_pair_cat = jnp.concatenate([W_pair[:HID], W_pair[HID:]], axis=1)
    freq_pad = jnp.pad(freq_bias, ((0, 0), (0, 128 - NUM_REL_CLS)))
    w_rel_pad = jnp.pad(W_relcls, ((0, 0), (0, 64 - NUM_REL_CLS)))
    b_rel_pad = jnp.pad(b_relcls, (0, 64 - NUM_REL_CLS)).reshape(1, 64)
    w_obj_pad = jnp.pad(W_objcls, ((0, 0), (0, 256 - NUM_OBJ_CLS)))
    b_obj_pad = jnp.pad(b_objcls, (0, 256 - NUM_OBJ_CLS)).reshape(1, 256)

    deg2 = _sc_deg(obj32, zeros128)
    fb = _sc_freq(subp, objp, obj_labels, freq_pad)

    h, ab = _tc_in_msg(roi_features, W_in, w_msg_cat)
    agg = _sc_agg(ab, subg, objg, zeros128)
    h, ab = _tc_upd_proj(h, agg, deg2, W_upd, w_msg_cat, 8)
    agg = _sc_agg(ab, subg, objg, zeros128)
    h, p2 = _tc_upd_proj(h, agg, deg2, W_upd, w_pair_cat, 2)

    relh = _sc_pair(p2, subpp, objpp)

    rel_out = _tc_rel(union_features, relh, fb, W_fuse, w_rel_pad, b_rel_pad)
    obj_out = _tc_obj(h, W_objproj, w_obj_pad, b_obj_pad)

    return (obj_out[:, :NUM_OBJ_CLS], rel_out[:, :NUM_REL_CLS])
